# baseline (speedup 1.0000x reference)
import functools

import jax
import jax.numpy as jnp
from jax.experimental import pallas as pl


def _sublane_tree(acc):
    return (((acc[0:1] + acc[4:5]) + (acc[2:3] + acc[6:7]))
            + ((acc[1:2] + acc[5:6]) + (acc[3:4] + acc[7:8])))


def _ms_kernel(x_ref, w_ref, out_ref):
    b = pl.program_id(0)

    def xbody(t, acc):
        for sl in range(4):
            tile = x_ref[sl, pl.ds(8 * t, 8), :]
            acc = acc + tile * tile
        return acc

    acc = jax.lax.fori_loop(
        0, 256, xbody, jnp.zeros((8, 128), jnp.float32))
    s = _sublane_tree(acc)

    r = s * jax.lax.rsqrt(s)
    r = jnp.where(s == jnp.inf, s, r)
    zero_signed = jax.lax.bitcast_convert_type(
        jax.lax.bitcast_convert_type(s, jnp.uint32) & jnp.uint32(0x80000000),
        jnp.float32)
    cn = jnp.where(s == 0.0, zero_signed, r)

    def wbody(g, _):
        blk = w_ref[pl.ds(128 * g, 128), :]
        mb = jnp.abs(blk) * cn
        tb = mb.T
        c = tb[0:8, :]
        for v in range(1, 16):
            c = c + tb[8 * v:8 * v + 8, :]
        p = _sublane_tree(c)
        prev = jnp.where(b == 0, jnp.zeros((1, 128), jnp.float32),
                         out_ref[pl.ds(g, 1), :])
        out_ref[pl.ds(g, 1), :] = prev + p
        return 0

    jax.lax.fori_loop(0, 64, wbody, 0)


def _compute_ms(W, X):
    out = pl.pallas_call(
        _ms_kernel,
        grid=(16,),
        in_specs=[
            pl.BlockSpec((4, 2048, 128), lambda b: (0, 0, b)),
            pl.BlockSpec((8192, 128), lambda b: (0, b)),
        ],
        out_specs=pl.BlockSpec((64, 128), lambda b: (0, 0)),
        out_shape=jax.ShapeDtypeStruct((64, 128), jnp.float32),
    )(X, W)
    return out.reshape(8192)


def kernel(W, X):
    ms = _compute_ms(W, X)
    sorted_idx = jnp.argsort(ms)
    return sorted_idx[:4096]


# speedup vs baseline: 2.2901x; 2.2901x over previous
import functools

import jax
import jax.numpy as jnp
from jax.experimental import pallas as pl


def _sublane_tree(acc):
    return (((acc[0:1] + acc[4:5]) + (acc[2:3] + acc[6:7]))
            + ((acc[1:2] + acc[5:6]) + (acc[3:4] + acc[7:8])))


def _ms_kernel(x_ref, w_ref, out_ref):
    b = pl.program_id(0)

    @pl.when(b == 0)
    def _():
        out_ref[...] = jnp.zeros((64, 128), jnp.float32)

    def xbody(t, acc):
        for sl in range(4):
            tile = x_ref[sl, pl.ds(8 * t, 8), :]
            acc = acc + tile * tile
        return acc

    acc = jax.lax.fori_loop(
        0, 256, xbody, jnp.zeros((8, 128), jnp.float32), unroll=8)
    s = _sublane_tree(acc)

    r = s * jax.lax.rsqrt(s)
    r = jnp.where(s == jnp.inf, s, r)
    zero_signed = jax.lax.bitcast_convert_type(
        jax.lax.bitcast_convert_type(s, jnp.uint32) & jnp.uint32(0x80000000),
        jnp.float32)
    cn = jnp.where(s == 0.0, zero_signed, r)

    def wbody(g, _):
        blk = w_ref[pl.ds(128 * g, 128), :]
        mb = jnp.abs(blk) * cn
        tb = mb.T
        c = tb[0:8, :]
        for v in range(1, 16):
            c = c + tb[8 * v:8 * v + 8, :]
        p = _sublane_tree(c)
        out_ref[pl.ds(g, 1), :] += p
        return 0

    jax.lax.fori_loop(0, 64, wbody, 0, unroll=4)


def _compute_ms(W, X):
    out = pl.pallas_call(
        _ms_kernel,
        grid=(16,),
        in_specs=[
            pl.BlockSpec((4, 2048, 128), lambda b: (0, 0, b)),
            pl.BlockSpec((8192, 128), lambda b: (0, b)),
        ],
        out_specs=pl.BlockSpec((64, 128), lambda b: (0, 0)),
        out_shape=jax.ShapeDtypeStruct((64, 128), jnp.float32),
    )(X, W)
    return out.reshape(8192)


def kernel(W, X):
    ms = _compute_ms(W, X)
    sorted_idx = jnp.argsort(ms)
    return sorted_idx[:4096]
